# baseline (speedup 1.0000x reference)
import functools

import jax
import jax.numpy as jnp
from jax import lax
from jax.experimental import pallas as pl
from jax.experimental.pallas import tpu as pltpu
from jax.experimental.pallas import tpu_sc as plsc

NC, NS = 2, 16
NW = NC * NS
BATCH, SEQ = 16384, 50
N = BATCH * SEQ
D = 64
DC = 2 * D
HID = 256
BLK = 128
NB = N // BLK
SPLIT = 8
NBS = NB // SPLIT
NS_ROWS = N // SPLIT
WB = NBS // NW
RING = 5
LAG = 2


def _sc_body(idx_hbm, tcat_hbm, emb_hbm, idx_v,
             b0, b1, b2, b3, b4, g0, g1, g2, g3, g4, w0, w1, w2, w3, w4):
    bufs = (b0, b1, b2, b3, b4)
    gs = (g0, g1, g2, g3, g4)
    ws = (w0, w1, w2, w3, w4)

    wid = lax.axis_index("s") * NC + lax.axis_index("c")
    wbase = wid * WB
    pltpu.sync_copy(idx_hbm.at[pl.ds(wbase, WB)], idx_v)

    def fire_gather(k, r):
        pltpu.async_copy(tcat_hbm.at[idx_v.at[k]], bufs[r], gs[r])

    def drain_g(r):
        pltpu.make_async_copy(tcat_hbm.at[idx_v.at[0]], bufs[r], gs[r]).wait()

    def fire_write(k, r):
        pltpu.async_copy(bufs[r], emb_hbm.at[pl.ds((wbase + k) * BLK, BLK)],
                         ws[r])

    def drain_w(r):
        pltpu.make_async_copy(tcat_hbm.at[idx_v.at[0]], bufs[r], ws[r]).wait()

    for k in range(RING):
        if k >= LAG:
            kd = k - LAG
            drain_g(kd % RING)
            fire_write(kd, kd % RING)
        fire_gather(k, k % RING)

    def loop_body(gi, carry):
        base = gi * RING
        for s in range(RING):
            k = base + s
            rd = (s - LAG) % RING
            drain_g(rd)
            fire_write(k - LAG, rd)
            drain_w(s)
            fire_gather(k, s)
        return carry

    lax.fori_loop(1, WB // RING, loop_body, 0)

    for j in range(LAG):
        k = WB - LAG + j
        r = k % RING
        drain_g(r)
        fire_write(k, r)
    for r in range(RING):
        drain_w(r)


_sc_gather = functools.partial(
    pl.kernel,
    out_type=jax.ShapeDtypeStruct((NS_ROWS, DC), jnp.float32),
    mesh=plsc.VectorSubcoreMesh(core_axis_name="c", subcore_axis_name="s"),
    scratch_types=(
        [pltpu.VMEM((WB, BLK), jnp.int32)]
        + [pltpu.VMEM((BLK, DC), jnp.float32)] * RING
        + [pltpu.SemaphoreType.DMA] * (2 * RING)
    ),
    compiler_params=pltpu.CompilerParams(use_tc_tiling_on_sc=False),
)(_sc_body)


NE = 1000000
PC = 16384


def _prep_body(t1_ref, t2_ref, out_ref):
    eye2 = jnp.concatenate(
        [jnp.eye(D, dtype=jnp.float32), jnp.zeros((D, D), jnp.float32)],
        axis=1)
    eye2b = jnp.concatenate(
        [jnp.zeros((D, D), jnp.float32), jnp.eye(D, dtype=jnp.float32)],
        axis=1)
    dn = (((0,), (0,)), ((), ()))
    a = jax.lax.dot_general(t1_ref[...], eye2, dn,
                            preferred_element_type=jnp.float32)
    b = jax.lax.dot_general(t2_ref[...], eye2b, dn,
                            preferred_element_type=jnp.float32)
    out_ref[...] = a + b


def _prep_tables(table1, table2):
    t1t = table1.T
    t2t = table2.T
    return pl.pallas_call(
        _prep_body,
        grid=(pl.cdiv(NE, PC),),
        in_specs=[
            pl.BlockSpec((D, PC), lambda i: (0, i)),
            pl.BlockSpec((D, PC), lambda i: (0, i)),
        ],
        out_specs=pl.BlockSpec((PC, DC), lambda i: (i, 0)),
        out_shape=jax.ShapeDtypeStruct((NE, DC), jnp.float32),
        compiler_params=pltpu.CompilerParams(
            fuse_transposed_lhs_in_matmul=True),
    )(t1t, t2t)


TCR = 10240
RBO = TCR // BLK
GRID = NS_ROWS // TCR


def _mlp_body(emb_ref, w1_ref, b1_ref, w2_ref, b2_ref, out_ref):
    x = emb_ref[...].astype(jnp.bfloat16)
    h = jnp.dot(x, w1_ref[...], preferred_element_type=jnp.float32)
    h = jnp.maximum(h + b1_ref[...], 0.0)
    o = jnp.sum(h * w2_ref[...], axis=1)
    out_ref[...] = o.reshape(RBO, BLK) + b2_ref[...]


def _mlp(emb, w1c, b1r, w2r, b2r):
    return pl.pallas_call(
        _mlp_body,
        grid=(GRID,),
        in_specs=[
            pl.BlockSpec((TCR, DC), lambda i: (i, 0)),
            pl.BlockSpec((DC, HID), lambda i: (0, 0)),
            pl.BlockSpec((1, HID), lambda i: (0, 0)),
            pl.BlockSpec((1, HID), lambda i: (0, 0)),
            pl.BlockSpec((1, 1), lambda i: (0, 0)),
        ],
        out_specs=pl.BlockSpec((RBO, BLK), lambda i: (i, 0)),
        out_shape=jax.ShapeDtypeStruct((NBS, BLK), jnp.float32),
    )(emb, w1c, b1r, w2r, b2r)


def kernel(input, table1, table2, W1, b1, W2, b2):
    idx = input.reshape(NB, BLK).astype(jnp.int32)
    tcat = _prep_tables(table1, table2)

    w1c = jnp.concatenate([W1, W1], axis=0).astype(jnp.bfloat16)
    b1r = b1.reshape(1, HID)
    w2r = W2.reshape(1, HID)
    b2r = b2.reshape(1, 1)

    outs = []
    for c in range(SPLIT):
        emb_c = _sc_gather(idx[c * NBS:(c + 1) * NBS], tcat)
        outs.append(_mlp(emb_c, w1c, b1r, w2r, b2r))
    out = jnp.concatenate(outs, axis=0)
    return out.reshape(BATCH, SEQ, 1)


# speedup vs baseline: 1.0163x; 1.0163x over previous
import functools

import jax
import jax.numpy as jnp
from jax import lax
from jax.experimental import pallas as pl
from jax.experimental.pallas import tpu as pltpu
from jax.experimental.pallas import tpu_sc as plsc

NC, NS = 2, 16
NW = NC * NS
BATCH, SEQ = 16384, 50
N = BATCH * SEQ
D = 64
DC = 2 * D
HID = 256
BLK = 128
NB = N // BLK
SPLIT = 4
NBS = NB // SPLIT
NS_ROWS = N // SPLIT
WB = NBS // NW
RING = 5
LAG = 2


def _sc_body(idx_hbm, tcat_hbm, emb_hbm, idx_v,
             b0, b1, b2, b3, b4, g0, g1, g2, g3, g4, w0, w1, w2, w3, w4):
    bufs = (b0, b1, b2, b3, b4)
    gs = (g0, g1, g2, g3, g4)
    ws = (w0, w1, w2, w3, w4)

    wid = lax.axis_index("s") * NC + lax.axis_index("c")
    wbase = wid * WB
    pltpu.sync_copy(idx_hbm.at[pl.ds(wbase, WB)], idx_v)

    def fire_gather(k, r):
        pltpu.async_copy(tcat_hbm.at[idx_v.at[k]], bufs[r], gs[r])

    def drain_g(r):
        pltpu.make_async_copy(tcat_hbm.at[idx_v.at[0]], bufs[r], gs[r]).wait()

    def fire_write(k, r):
        pltpu.async_copy(bufs[r], emb_hbm.at[pl.ds((wbase + k) * BLK, BLK)],
                         ws[r])

    def drain_w(r):
        pltpu.make_async_copy(tcat_hbm.at[idx_v.at[0]], bufs[r], ws[r]).wait()

    for k in range(RING):
        if k >= LAG:
            kd = k - LAG
            drain_g(kd % RING)
            fire_write(kd, kd % RING)
        fire_gather(k, k % RING)

    def loop_body(gi, carry):
        base = gi * RING
        for s in range(RING):
            k = base + s
            rd = (s - LAG) % RING
            drain_g(rd)
            fire_write(k - LAG, rd)
            drain_w(s)
            fire_gather(k, s)
        return carry

    lax.fori_loop(1, WB // RING, loop_body, 0)

    for j in range(LAG):
        k = WB - LAG + j
        r = k % RING
        drain_g(r)
        fire_write(k, r)
    for r in range(RING):
        drain_w(r)


_sc_gather = functools.partial(
    pl.kernel,
    out_type=jax.ShapeDtypeStruct((NS_ROWS, DC), jnp.float32),
    mesh=plsc.VectorSubcoreMesh(core_axis_name="c", subcore_axis_name="s"),
    scratch_types=(
        [pltpu.VMEM((WB, BLK), jnp.int32)]
        + [pltpu.VMEM((BLK, DC), jnp.float32)] * RING
        + [pltpu.SemaphoreType.DMA] * (2 * RING)
    ),
    compiler_params=pltpu.CompilerParams(use_tc_tiling_on_sc=False),
)(_sc_body)


NE = 1000000
PC = 16384


def _prep_body(t1_ref, t2_ref, out_ref):
    eye2 = jnp.concatenate(
        [jnp.eye(D, dtype=jnp.float32), jnp.zeros((D, D), jnp.float32)],
        axis=1)
    eye2b = jnp.concatenate(
        [jnp.zeros((D, D), jnp.float32), jnp.eye(D, dtype=jnp.float32)],
        axis=1)
    dn = (((0,), (0,)), ((), ()))
    a = jax.lax.dot_general(t1_ref[...], eye2, dn,
                            preferred_element_type=jnp.float32)
    b = jax.lax.dot_general(t2_ref[...], eye2b, dn,
                            preferred_element_type=jnp.float32)
    out_ref[...] = a + b


def _prep_tables(table1, table2):
    t1t = table1.T
    t2t = table2.T
    return pl.pallas_call(
        _prep_body,
        grid=(pl.cdiv(NE, PC),),
        in_specs=[
            pl.BlockSpec((D, PC), lambda i: (0, i)),
            pl.BlockSpec((D, PC), lambda i: (0, i)),
        ],
        out_specs=pl.BlockSpec((PC, DC), lambda i: (i, 0)),
        out_shape=jax.ShapeDtypeStruct((NE, DC), jnp.float32),
        compiler_params=pltpu.CompilerParams(
            fuse_transposed_lhs_in_matmul=True),
    )(t1t, t2t)


TCR = 20480
RBO = TCR // BLK
GRID = NS_ROWS // TCR


def _mlp_body(emb_ref, w1_ref, b1_ref, w2_ref, b2_ref, out_ref):
    x = emb_ref[...].astype(jnp.bfloat16)
    h = jnp.dot(x, w1_ref[...], preferred_element_type=jnp.float32)
    h = jnp.maximum(h + b1_ref[...], 0.0)
    o = jnp.sum(h * w2_ref[...], axis=1)
    out_ref[...] = o.reshape(RBO, BLK) + b2_ref[...]


def _mlp(emb, w1c, b1r, w2r, b2r):
    return pl.pallas_call(
        _mlp_body,
        grid=(GRID,),
        in_specs=[
            pl.BlockSpec((TCR, DC), lambda i: (i, 0)),
            pl.BlockSpec((DC, HID), lambda i: (0, 0)),
            pl.BlockSpec((1, HID), lambda i: (0, 0)),
            pl.BlockSpec((1, HID), lambda i: (0, 0)),
            pl.BlockSpec((1, 1), lambda i: (0, 0)),
        ],
        out_specs=pl.BlockSpec((RBO, BLK), lambda i: (i, 0)),
        out_shape=jax.ShapeDtypeStruct((NBS, BLK), jnp.float32),
    )(emb, w1c, b1r, w2r, b2r)


def kernel(input, table1, table2, W1, b1, W2, b2):
    idx = input.reshape(NB, BLK).astype(jnp.int32)
    tcat = _prep_tables(table1, table2)

    w1c = jnp.concatenate([W1, W1], axis=0).astype(jnp.bfloat16)
    b1r = b1.reshape(1, HID)
    w2r = W2.reshape(1, HID)
    b2r = b2.reshape(1, 1)

    outs = []
    for c in range(SPLIT):
        emb_c = _sc_gather(idx[c * NBS:(c + 1) * NBS], tcat)
        outs.append(_mlp(emb_c, w1c, b1r, w2r, b2r))
    out = jnp.concatenate(outs, axis=0)
    return out.reshape(BATCH, SEQ, 1)


# speedup vs baseline: 1.0276x; 1.0111x over previous
import functools

import jax
import jax.numpy as jnp
from jax import lax
from jax.experimental import pallas as pl
from jax.experimental.pallas import tpu as pltpu
from jax.experimental.pallas import tpu_sc as plsc

NC, NS = 2, 16
NW = NC * NS
BATCH, SEQ = 16384, 50
N = BATCH * SEQ
D = 64
DC = 2 * D
HID = 256
BLK = 128
NB = N // BLK
SPLIT = 4
NBS = NB // SPLIT
NS_ROWS = N // SPLIT
WB = NBS // NW
RING = 5
LAG = 2


def _sc_body(idx_hbm, tcat_hbm, emb_hbm, idx_v,
             b0, b1, b2, b3, b4, g0, g1, g2, g3, g4, w0, w1, w2, w3, w4):
    bufs = (b0, b1, b2, b3, b4)
    gs = (g0, g1, g2, g3, g4)
    ws = (w0, w1, w2, w3, w4)

    wid = lax.axis_index("s") * NC + lax.axis_index("c")
    wbase = wid * WB
    pltpu.sync_copy(idx_hbm.at[pl.ds(wbase, WB)], idx_v)

    def fire_gather(k, r):
        pltpu.async_copy(tcat_hbm.at[idx_v.at[k]], bufs[r], gs[r])

    def drain_g(r):
        pltpu.make_async_copy(tcat_hbm.at[idx_v.at[0]], bufs[r], gs[r]).wait()

    def fire_write(k, r):
        pltpu.async_copy(bufs[r], emb_hbm.at[pl.ds((wbase + k) * BLK, BLK)],
                         ws[r])

    def drain_w(r):
        pltpu.make_async_copy(tcat_hbm.at[idx_v.at[0]], bufs[r], ws[r]).wait()

    for k in range(RING):
        if k >= LAG:
            kd = k - LAG
            drain_g(kd % RING)
            fire_write(kd, kd % RING)
        fire_gather(k, k % RING)

    def loop_body(gi, carry):
        base = gi * RING
        for s in range(RING):
            k = base + s
            rd = (s - LAG) % RING
            drain_g(rd)
            fire_write(k - LAG, rd)
            drain_w(s)
            fire_gather(k, s)
        return carry

    lax.fori_loop(1, WB // RING, loop_body, 0)

    for j in range(LAG):
        k = WB - LAG + j
        r = k % RING
        drain_g(r)
        fire_write(k, r)
    for r in range(RING):
        drain_w(r)


_sc_gather = functools.partial(
    pl.kernel,
    out_type=jax.ShapeDtypeStruct((NS_ROWS, DC), jnp.float32),
    mesh=plsc.VectorSubcoreMesh(core_axis_name="c", subcore_axis_name="s"),
    scratch_types=(
        [pltpu.VMEM((WB, BLK), jnp.int32)]
        + [pltpu.VMEM((BLK, DC), jnp.float32)] * RING
        + [pltpu.SemaphoreType.DMA] * (2 * RING)
    ),
    compiler_params=pltpu.CompilerParams(use_tc_tiling_on_sc=False),
)(_sc_body)


NE = 1000000
PC = 16384


def _prep_body(t1_ref, t2_ref, out_ref):
    eye2 = jnp.concatenate(
        [jnp.eye(D, dtype=jnp.float32), jnp.zeros((D, D), jnp.float32)],
        axis=1)
    eye2b = jnp.concatenate(
        [jnp.zeros((D, D), jnp.float32), jnp.eye(D, dtype=jnp.float32)],
        axis=1)
    dn = (((0,), (0,)), ((), ()))
    a = jax.lax.dot_general(t1_ref[...], eye2, dn,
                            preferred_element_type=jnp.float32)
    b = jax.lax.dot_general(t2_ref[...], eye2b, dn,
                            preferred_element_type=jnp.float32)
    out_ref[...] = a + b


def _prep_tables(table1, table2):
    t1t = table1.T
    t2t = table2.T
    return pl.pallas_call(
        _prep_body,
        grid=(pl.cdiv(NE, PC),),
        in_specs=[
            pl.BlockSpec((D, PC), lambda i: (0, i)),
            pl.BlockSpec((D, PC), lambda i: (0, i)),
        ],
        out_specs=pl.BlockSpec((PC, DC), lambda i: (i, 0)),
        out_shape=jax.ShapeDtypeStruct((NE, DC), jnp.float32),
        compiler_params=pltpu.CompilerParams(
            fuse_transposed_lhs_in_matmul=True),
    )(t1t, t2t)


TCR = 16384
RBO = TCR // BLK
GRID = NS_ROWS // TCR


def _mlp_body(emb_ref, w1_ref, b1_ref, w2_ref, b2_ref, out_ref):
    x = emb_ref[...].astype(jnp.bfloat16)
    h = jnp.dot(x, w1_ref[...], preferred_element_type=jnp.float32)
    h = jnp.maximum(h + b1_ref[...], 0.0)
    o = jnp.sum(h * w2_ref[...], axis=1)
    out_ref[...] = o.reshape(RBO, BLK) + b2_ref[...]


def _mlp(emb, w1c, b1r, w2r, b2r):
    return pl.pallas_call(
        _mlp_body,
        grid=(GRID,),
        in_specs=[
            pl.BlockSpec((TCR, DC), lambda i: (i, 0)),
            pl.BlockSpec((DC, HID), lambda i: (0, 0)),
            pl.BlockSpec((1, HID), lambda i: (0, 0)),
            pl.BlockSpec((1, HID), lambda i: (0, 0)),
            pl.BlockSpec((1, 1), lambda i: (0, 0)),
        ],
        out_specs=pl.BlockSpec((RBO, BLK), lambda i: (i, 0)),
        out_shape=jax.ShapeDtypeStruct((NBS, BLK), jnp.float32),
    )(emb, w1c, b1r, w2r, b2r)


def kernel(input, table1, table2, W1, b1, W2, b2):
    idx = input.reshape(NB, BLK).astype(jnp.int32)
    tcat = _prep_tables(table1, table2)

    w1c = jnp.concatenate([W1, W1], axis=0).astype(jnp.bfloat16)
    b1r = b1.reshape(1, HID)
    w2r = W2.reshape(1, HID)
    b2r = b2.reshape(1, 1)

    outs = []
    for c in range(SPLIT):
        emb_c = _sc_gather(idx[c * NBS:(c + 1) * NBS], tcat)
        outs.append(_mlp(emb_c, w1c, b1r, w2r, b2r))
    out = jnp.concatenate(outs, axis=0)
    return out.reshape(BATCH, SEQ, 1)
